# baseline (speedup 1.0000x reference)
import functools

import jax
import jax.numpy as jnp
from jax import lax
from jax.experimental import pallas as pl
from jax.experimental.pallas import tpu as pltpu
from jax.experimental.pallas import tpu_sc as plsc

DEPTH = 4
H = 256
HW = H // 2
FA = 128
FB_IN = 144
N = 10000
E = 320000
MAX_NB = 32
APM = 20
N_MOLS = N // APM

NW = 32
N_PAD = 10240
A_PER_W = N_PAD // NW
GS = 4
NB_NEI = 4
A_FAST = 480
A_SLOW = 160
FAST_C = 1
NG_NEI = A_PER_W // GS
BONDS_PER_W = E // NW
BC = 80
NG_TR = BONDS_PER_W // BC
F32 = jnp.float32
BF16 = jnp.bfloat16

_HI_MASK = -65536
_LO_MASK = 65535
_RND = 32768

_mesh = plsc.VectorSubcoreMesh(core_axis_name="c", subcore_axis_name="s")


def _wid():
    return lax.axis_index("s") * 2 + lax.axis_index("c")


@functools.partial(
    pl.kernel,
    out_type=jax.ShapeDtypeStruct((N_PAD * MAX_NB, HW), jnp.int32),
    mesh=_mesh,
    scratch_types=[
        pltpu.VMEM((A_FAST * MAX_NB,), jnp.int32),
        pltpu.VMEM((NB_NEI, GS * MAX_NB, HW), jnp.int32),
    ] + [pltpu.SemaphoreType.DMA] * (2 * NB_NEI),
)
def _sc_nei(msgp_hbm, a2b_hbm, out_hbm, idx_all, rows_v, *sems):
    c = lax.axis_index("c")
    s = lax.axis_index("s")
    sem_g = sems[:NB_NEI]
    sem_o = sems[NB_NEI:]
    cn = GS * MAX_NB
    on_fast = c == FAST_C
    acnt = jnp.where(on_fast, A_FAST, A_SLOW)
    abase = jnp.where(on_fast, s * A_FAST,
                      (NW // 2) * A_FAST + s * A_SLOW)
    ng = acnt // GS
    pltpu.sync_copy(a2b_hbm.at[pl.ds(abase * MAX_NB, A_FAST * MAX_NB)],
                    idx_all)
    for b in range(NB_NEI):
        pltpu.async_copy(msgp_hbm.at[idx_all.at[pl.ds(b * cn, cn)]],
                         rows_v.at[b], sem_g[b])

    def rnd(k, carry):
        for b in range(NB_NEI):
            g = NB_NEI * k + b
            pltpu.make_async_copy(
                msgp_hbm.at[idx_all.at[pl.ds(0, cn)]],
                rows_v.at[b], sem_g[b]).wait()
            pltpu.async_copy(
                rows_v.at[b],
                out_hbm.at[pl.ds(abase * MAX_NB + g * cn, cn)],
                sem_o[b])
            pltpu.make_async_copy(
                rows_v.at[b], out_hbm.at[pl.ds(0, cn)], sem_o[b]).wait()

            @pl.when(g + NB_NEI < ng)
            def _():
                pltpu.async_copy(
                    msgp_hbm.at[idx_all.at[pl.ds((g + NB_NEI) * cn, cn)]],
                    rows_v.at[b], sem_g[b])
        return carry

    lax.fori_loop(0, ng // NB_NEI, rnd, 0)


@functools.partial(
    pl.kernel,
    out_type=[
        jax.ShapeDtypeStruct((E, HW), jnp.int32),
        jax.ShapeDtypeStruct((E, HW), jnp.int32),
    ],
    mesh=_mesh,
    scratch_types=[
        pltpu.VMEM((BONDS_PER_W,), jnp.int32),
        pltpu.VMEM((BONDS_PER_W,), jnp.int32),
        pltpu.VMEM((2, BC, HW), jnp.int32),
        pltpu.VMEM((2, BC, HW), jnp.int32),
        pltpu.SemaphoreType.DMA,
        pltpu.SemaphoreType.DMA,
        pltpu.SemaphoreType.DMA,
        pltpu.SemaphoreType.DMA,
        pltpu.SemaphoreType.DMA,
        pltpu.SemaphoreType.DMA,
        pltpu.SemaphoreType.DMA,
        pltpu.SemaphoreType.DMA,
    ],
)
def _sc_tr(amsgp_hbm, msgp_hbm, b2a_hbm, b2revb_hbm,
           t_hbm, r_hbm, idxa_all, idxr_all, buf_t, buf_r,
           sem_t0, sem_t1, sem_r0, sem_r1,
           sem_ot0, sem_ot1, sem_or0, sem_or1):
    w = _wid()
    sem_t = (sem_t0, sem_t1)
    sem_r = (sem_r0, sem_r1)
    sem_ot = (sem_ot0, sem_ot1)
    sem_or = (sem_or0, sem_or1)
    pltpu.sync_copy(b2a_hbm.at[pl.ds(w * BONDS_PER_W, BONDS_PER_W)], idxa_all)
    pltpu.sync_copy(b2revb_hbm.at[pl.ds(w * BONDS_PER_W, BONDS_PER_W)],
                    idxr_all)
    for b in range(2):
        pltpu.async_copy(amsgp_hbm.at[idxa_all.at[pl.ds(b * BC, BC)]],
                         buf_t.at[b], sem_t[b])
        pltpu.async_copy(msgp_hbm.at[idxr_all.at[pl.ds(b * BC, BC)]],
                         buf_r.at[b], sem_r[b])

    def one_group(g, b):
        base = pl.ds(w * BONDS_PER_W + g * BC, BC)
        pltpu.make_async_copy(
            amsgp_hbm.at[idxa_all.at[pl.ds(0, BC)]],
            buf_t.at[b], sem_t[b]).wait()
        pltpu.make_async_copy(
            msgp_hbm.at[idxr_all.at[pl.ds(0, BC)]],
            buf_r.at[b], sem_r[b]).wait()
        pltpu.async_copy(buf_t.at[b], t_hbm.at[base], sem_ot[b])
        pltpu.async_copy(buf_r.at[b], r_hbm.at[base], sem_or[b])
        pltpu.make_async_copy(
            buf_t.at[b], t_hbm.at[pl.ds(0, BC)], sem_ot[b]).wait()
        pltpu.make_async_copy(
            buf_r.at[b], r_hbm.at[pl.ds(0, BC)], sem_or[b]).wait()

        if isinstance(g, int) and g + 2 >= NG_TR:
            return

        @pl.when(g + 2 < NG_TR)
        def _():
            nxt = pl.ds((g + 2) * BC, BC)
            pltpu.async_copy(amsgp_hbm.at[idxa_all.at[nxt]],
                             buf_t.at[b], sem_t[b])
            pltpu.async_copy(msgp_hbm.at[idxr_all.at[nxt]],
                             buf_r.at[b], sem_r[b])

    def pair(k, carry):
        for b in range(2):
            one_group(2 * k + b, b)
        return carry

    lax.fori_loop(0, NG_TR // 2, pair, 0)
    if NG_TR % 2:
        one_group(NG_TR - 1, 0)


def _pack(lo, hi):
    wl = lax.bitcast_convert_type(lo, jnp.int32)
    wh = lax.bitcast_convert_type(hi, jnp.int32)
    return (((wl + _RND) >> 16) & _LO_MASK) | ((wh + _RND) & _HI_MASK)


def _unpack(w):
    lo = lax.bitcast_convert_type(w << 16, F32)
    hi = lax.bitcast_convert_type(w & _HI_MASK, F32)
    return lo, hi


RB = 2560


def _mm_init_body(fb_ref, wi_ref, inpp_ref, msgp_ref):
    x = jnp.dot(fb_ref[...], wi_ref[...], preferred_element_type=F32)
    inpp_ref[...] = _pack(x[:, :HW], x[:, HW:])
    m = jnp.maximum(x, 0.0)
    msgp_ref[...] = _pack(m[:, :HW], m[:, HW:])


_mm_init = pl.pallas_call(
    _mm_init_body,
    grid=(E // RB,),
    in_specs=[
        pl.BlockSpec((RB, FB_IN), lambda i: (i, 0)),
        pl.BlockSpec((FB_IN, H), lambda i: (0, 0)),
    ],
    out_specs=[
        pl.BlockSpec((RB, HW), lambda i: (i, 0)),
        pl.BlockSpec((RB, HW), lambda i: (i, 0)),
    ],
    out_shape=[
        jax.ShapeDtypeStruct((E, HW), jnp.int32),
        jax.ShapeDtypeStruct((E, HW), jnp.int32),
    ],
)


def _mm_iter_body(t_ref, r_ref, inpp_ref, wht_ref, whb_ref, msgp_ref):
    t_lo, t_hi = _unpack(t_ref[...])
    r_lo, r_hi = _unpack(r_ref[...])
    d_lo = (t_lo - r_lo).astype(BF16)
    d_hi = (t_hi - r_hi).astype(BF16)
    x = jnp.dot(d_lo, wht_ref[...], preferred_element_type=F32)
    x = x + jnp.dot(d_hi, whb_ref[...], preferred_element_type=F32)
    i_lo, i_hi = _unpack(inpp_ref[...])
    o_lo = jnp.maximum(i_lo + x[:, :HW], 0.0)
    o_hi = jnp.maximum(i_hi + x[:, HW:], 0.0)
    msgp_ref[...] = _pack(o_lo, o_hi)


_mm_iter = pl.pallas_call(
    _mm_iter_body,
    grid=(E // RB,),
    in_specs=[
        pl.BlockSpec((RB, HW), lambda i: (i, 0)),
        pl.BlockSpec((RB, HW), lambda i: (i, 0)),
        pl.BlockSpec((RB, HW), lambda i: (i, 0)),
        pl.BlockSpec((HW, H), lambda i: (0, 0)),
        pl.BlockSpec((HW, H), lambda i: (0, 0)),
    ],
    out_specs=pl.BlockSpec((RB, HW), lambda i: (i, 0)),
    out_shape=jax.ShapeDtypeStruct((E, HW), jnp.int32),
)

AT = 256


def _tc_sum_body(nei_ref, p_ref, amsgp_ref):
    lo, hi = _unpack(nei_ref[...])
    p = p_ref[...]
    s_lo = jnp.dot(p, lo.astype(BF16), preferred_element_type=F32)
    s_hi = jnp.dot(p, hi.astype(BF16), preferred_element_type=F32)
    amsgp_ref[...] = _pack(s_lo, s_hi)


_tc_sum = pl.pallas_call(
    _tc_sum_body,
    grid=(N_PAD // AT,),
    in_specs=[
        pl.BlockSpec((AT * MAX_NB, HW), lambda i: (i, 0)),
        pl.BlockSpec((AT, AT * MAX_NB), lambda i: (0, 0)),
    ],
    out_specs=pl.BlockSpec((AT, HW), lambda i: (i, 0)),
    out_shape=jax.ShapeDtypeStruct((N_PAD, HW), jnp.int32),
)

ATF = 160


def _tc_final_body(nei_ref, fa_ref, ps_ref, pm_ref, woa_ref, wol_ref,
                   woh_ref, ah_ref, mv_ref):
    lo, hi = _unpack(nei_ref[...])
    ps = ps_ref[...]
    s_lo = jnp.dot(ps, lo.astype(BF16), preferred_element_type=F32)
    s_hi = jnp.dot(ps, hi.astype(BF16), preferred_element_type=F32)
    x = jnp.dot(fa_ref[...], woa_ref[...], preferred_element_type=F32)
    x = x + jnp.dot(s_lo, wol_ref[...], preferred_element_type=F32)
    x = x + jnp.dot(s_hi, woh_ref[...], preferred_element_type=F32)
    ah = jnp.maximum(x, 0.0)
    ah_ref[...] = ah
    mv_ref[...] = jnp.dot(pm_ref[...], ah, preferred_element_type=F32)


_tc_final = pl.pallas_call(
    _tc_final_body,
    grid=(N_PAD // ATF,),
    in_specs=[
        pl.BlockSpec((ATF * MAX_NB, HW), lambda i: (i, 0)),
        pl.BlockSpec((ATF, FA), lambda i: (i, 0)),
        pl.BlockSpec((ATF, ATF * MAX_NB), lambda i: (0, 0)),
        pl.BlockSpec((ATF // APM, ATF), lambda i: (0, 0)),
        pl.BlockSpec((FA, H), lambda i: (0, 0)),
        pl.BlockSpec((HW, H), lambda i: (0, 0)),
        pl.BlockSpec((HW, H), lambda i: (0, 0)),
    ],
    out_specs=[
        pl.BlockSpec((ATF, H), lambda i: (i, 0)),
        pl.BlockSpec((ATF // APM, H), lambda i: (i, 0)),
    ],
    out_shape=[
        jax.ShapeDtypeStruct((N_PAD, H), F32),
        jax.ShapeDtypeStruct((N_PAD // APM, H), F32),
    ],
)


def kernel(f_atoms, f_bonds, f_mols, a2b, b2a, b2revb, atoms_per_mol, W_i, W_h, W_o):
    del f_mols, atoms_per_mol
    a2b_flat = jnp.concatenate(
        [a2b, jnp.zeros((N_PAD - N + A_FAST, MAX_NB), jnp.int32)], axis=0
    ).reshape(-1)
    f_atoms_pad = jnp.concatenate(
        [f_atoms, jnp.zeros((N_PAD - N, FA), F32)], axis=0)
    wh_top = W_h[:HW].astype(BF16)
    wh_bot = W_h[HW:].astype(BF16)
    gsum = lax.broadcasted_iota(jnp.int32, (AT, AT * MAX_NB), 1) // MAX_NB
    p_sum = (gsum == lax.broadcasted_iota(jnp.int32, (AT, AT * MAX_NB), 0)
             ).astype(BF16)
    gfin = lax.broadcasted_iota(jnp.int32, (ATF, ATF * MAX_NB), 1) // MAX_NB
    p_fin = (gfin == lax.broadcasted_iota(jnp.int32, (ATF, ATF * MAX_NB), 0)
             ).astype(BF16)
    gmol = lax.broadcasted_iota(jnp.int32, (ATF // APM, ATF), 1) // APM
    p_mol = jnp.where(
        gmol == lax.broadcasted_iota(jnp.int32, (ATF // APM, ATF), 0),
        1.0 / APM, 0.0).astype(F32)

    inpp, msgp = _mm_init(f_bonds, W_i)
    for _ in range(DEPTH - 1):
        nei = _sc_nei(msgp, a2b_flat)
        amsgp = _tc_sum(nei, p_sum)
        t, r = _sc_tr(amsgp, msgp, b2a, b2revb)
        msgp = _mm_iter(t, r, inpp, wh_top, wh_bot)
    nei = _sc_nei(msgp, a2b_flat)
    atom_h, mol_vecs = _tc_final(
        nei, f_atoms_pad, p_fin, p_mol, W_o[:FA], W_o[FA:FA + HW],
        W_o[FA + HW:])
    return (mol_vecs[:N_MOLS], atom_h[:N])


# speedup vs baseline: 1.0938x; 1.0938x over previous
import functools

import jax
import jax.numpy as jnp
from jax import lax
from jax.experimental import pallas as pl
from jax.experimental.pallas import tpu as pltpu
from jax.experimental.pallas import tpu_sc as plsc

DEPTH = 4
H = 256
HW = H // 2
FA = 128
FB_IN = 144
N = 10000
E = 320000
MAX_NB = 32
APM = 20
N_MOLS = N // APM

NW = 32
N_PAD = 10240
A_PER_W = N_PAD // NW
GS = 4
NB_NEI = 4
A_FAST = 320
A_SLOW = 320
FAST_C = 0
NG_NEI = A_PER_W // GS
BONDS_PER_W = E // NW
BC = 80
NG_TR = BONDS_PER_W // BC
F32 = jnp.float32
BF16 = jnp.bfloat16

_HI_MASK = -65536
_LO_MASK = 65535
_RND = 32768

_mesh = plsc.VectorSubcoreMesh(core_axis_name="c", subcore_axis_name="s")


def _wid():
    return lax.axis_index("s") * 2 + lax.axis_index("c")


@functools.partial(
    pl.kernel,
    out_type=jax.ShapeDtypeStruct((N_PAD * MAX_NB, HW), jnp.int32),
    mesh=_mesh,
    scratch_types=[
        pltpu.VMEM((A_FAST * MAX_NB,), jnp.int32),
        pltpu.VMEM((NB_NEI, GS * MAX_NB, HW), jnp.int32),
    ] + [pltpu.SemaphoreType.DMA] * (2 * NB_NEI),
)
def _sc_nei(msgp_hbm, a2b_hbm, out_hbm, idx_all, rows_v, *sems):
    c = lax.axis_index("c")
    s = lax.axis_index("s")
    sem_g = sems[:NB_NEI]
    sem_o = sems[NB_NEI:]
    cn = GS * MAX_NB
    on_fast = c == FAST_C
    acnt = jnp.where(on_fast, A_FAST, A_SLOW)
    abase = jnp.where(on_fast, s * A_FAST,
                      (NW // 2) * A_FAST + s * A_SLOW)
    ng = acnt // GS
    pltpu.sync_copy(a2b_hbm.at[pl.ds(abase * MAX_NB, A_FAST * MAX_NB)],
                    idx_all)
    for b in range(NB_NEI):
        pltpu.async_copy(msgp_hbm.at[idx_all.at[pl.ds(b * cn, cn)]],
                         rows_v.at[b], sem_g[b])

    def rnd(k, carry):
        for b in range(NB_NEI):
            g = NB_NEI * k + b
            pltpu.make_async_copy(
                msgp_hbm.at[idx_all.at[pl.ds(0, cn)]],
                rows_v.at[b], sem_g[b]).wait()
            pltpu.async_copy(
                rows_v.at[b],
                out_hbm.at[pl.ds(abase * MAX_NB + g * cn, cn)],
                sem_o[b])
            pltpu.make_async_copy(
                rows_v.at[b], out_hbm.at[pl.ds(0, cn)], sem_o[b]).wait()

            @pl.when(g + NB_NEI < ng)
            def _():
                pltpu.async_copy(
                    msgp_hbm.at[idx_all.at[pl.ds((g + NB_NEI) * cn, cn)]],
                    rows_v.at[b], sem_g[b])
        return carry

    lax.fori_loop(0, ng // NB_NEI, rnd, 0)


NB_TR = 5


@functools.partial(
    pl.kernel,
    out_type=jax.ShapeDtypeStruct((E, HW), jnp.int32),
    mesh=_mesh,
    scratch_types=[
        pltpu.VMEM((BONDS_PER_W,), jnp.int32),
        pltpu.VMEM((NB_TR, BC, HW), jnp.int32),
    ] + [pltpu.SemaphoreType.DMA] * (2 * NB_TR),
)
def _sc_gather(tab_hbm, idx_hbm, out_hbm, idx_all, bufs, *sems):
    w = _wid()
    sem_g = sems[:NB_TR]
    sem_o = sems[NB_TR:]
    pltpu.sync_copy(idx_hbm.at[pl.ds(w * BONDS_PER_W, BONDS_PER_W)], idx_all)
    for b in range(NB_TR):
        pltpu.async_copy(tab_hbm.at[idx_all.at[pl.ds(b * BC, BC)]],
                         bufs.at[b], sem_g[b])

    def rnd(k, carry):
        for b in range(NB_TR):
            g = NB_TR * k + b
            pltpu.make_async_copy(
                tab_hbm.at[idx_all.at[pl.ds(0, BC)]],
                bufs.at[b], sem_g[b]).wait()
            pltpu.async_copy(
                bufs.at[b],
                out_hbm.at[pl.ds(w * BONDS_PER_W + g * BC, BC)], sem_o[b])
            pltpu.make_async_copy(
                bufs.at[b], out_hbm.at[pl.ds(0, BC)], sem_o[b]).wait()

            @pl.when(g + NB_TR < NG_TR)
            def _():
                pltpu.async_copy(
                    tab_hbm.at[idx_all.at[pl.ds((g + NB_TR) * BC, BC)]],
                    bufs.at[b], sem_g[b])
        return carry

    lax.fori_loop(0, NG_TR // NB_TR, rnd, 0)


def _pack(lo, hi):
    wl = lax.bitcast_convert_type(lo, jnp.int32)
    wh = lax.bitcast_convert_type(hi, jnp.int32)
    return (((wl + _RND) >> 16) & _LO_MASK) | ((wh + _RND) & _HI_MASK)


def _unpack(w):
    lo = lax.bitcast_convert_type(w << 16, F32)
    hi = lax.bitcast_convert_type(w & _HI_MASK, F32)
    return lo, hi


RB = 2560


def _mm_init_body(fb_ref, wi_ref, inpp_ref, msgp_ref):
    x = jnp.dot(fb_ref[...], wi_ref[...], preferred_element_type=F32)
    inpp_ref[...] = _pack(x[:, :HW], x[:, HW:])
    m = jnp.maximum(x, 0.0)
    msgp_ref[...] = _pack(m[:, :HW], m[:, HW:])


_mm_init = pl.pallas_call(
    _mm_init_body,
    grid=(E // RB,),
    in_specs=[
        pl.BlockSpec((RB, FB_IN), lambda i: (i, 0)),
        pl.BlockSpec((FB_IN, H), lambda i: (0, 0)),
    ],
    out_specs=[
        pl.BlockSpec((RB, HW), lambda i: (i, 0)),
        pl.BlockSpec((RB, HW), lambda i: (i, 0)),
    ],
    out_shape=[
        jax.ShapeDtypeStruct((E, HW), jnp.int32),
        jax.ShapeDtypeStruct((E, HW), jnp.int32),
    ],
)


def _mm_iter_body(t_ref, r_ref, inpp_ref, wht_ref, whb_ref, msgp_ref):
    t_lo, t_hi = _unpack(t_ref[...])
    r_lo, r_hi = _unpack(r_ref[...])
    d_lo = (t_lo - r_lo).astype(BF16)
    d_hi = (t_hi - r_hi).astype(BF16)
    x = jnp.dot(d_lo, wht_ref[...], preferred_element_type=F32)
    x = x + jnp.dot(d_hi, whb_ref[...], preferred_element_type=F32)
    i_lo, i_hi = _unpack(inpp_ref[...])
    o_lo = jnp.maximum(i_lo + x[:, :HW], 0.0)
    o_hi = jnp.maximum(i_hi + x[:, HW:], 0.0)
    msgp_ref[...] = _pack(o_lo, o_hi)


_mm_iter = pl.pallas_call(
    _mm_iter_body,
    grid=(E // RB,),
    in_specs=[
        pl.BlockSpec((RB, HW), lambda i: (i, 0)),
        pl.BlockSpec((RB, HW), lambda i: (i, 0)),
        pl.BlockSpec((RB, HW), lambda i: (i, 0)),
        pl.BlockSpec((HW, H), lambda i: (0, 0)),
        pl.BlockSpec((HW, H), lambda i: (0, 0)),
    ],
    out_specs=pl.BlockSpec((RB, HW), lambda i: (i, 0)),
    out_shape=jax.ShapeDtypeStruct((E, HW), jnp.int32),
)

AT = 256


def _tc_sum_body(nei_ref, p_ref, amsgp_ref):
    lo, hi = _unpack(nei_ref[...])
    p = p_ref[...]
    s_lo = jnp.dot(p, lo.astype(BF16), preferred_element_type=F32)
    s_hi = jnp.dot(p, hi.astype(BF16), preferred_element_type=F32)
    amsgp_ref[...] = _pack(s_lo, s_hi)


_tc_sum = pl.pallas_call(
    _tc_sum_body,
    grid=(N_PAD // AT,),
    in_specs=[
        pl.BlockSpec((AT * MAX_NB, HW), lambda i: (i, 0)),
        pl.BlockSpec((AT, AT * MAX_NB), lambda i: (0, 0)),
    ],
    out_specs=pl.BlockSpec((AT, HW), lambda i: (i, 0)),
    out_shape=jax.ShapeDtypeStruct((N_PAD, HW), jnp.int32),
)

ATF = 160


def _tc_final_body(nei_ref, fa_ref, ps_ref, pm_ref, woa_ref, wol_ref,
                   woh_ref, ah_ref, mv_ref):
    lo, hi = _unpack(nei_ref[...])
    ps = ps_ref[...]
    s_lo = jnp.dot(ps, lo.astype(BF16), preferred_element_type=F32)
    s_hi = jnp.dot(ps, hi.astype(BF16), preferred_element_type=F32)
    x = jnp.dot(fa_ref[...], woa_ref[...], preferred_element_type=F32)
    x = x + jnp.dot(s_lo, wol_ref[...], preferred_element_type=F32)
    x = x + jnp.dot(s_hi, woh_ref[...], preferred_element_type=F32)
    ah = jnp.maximum(x, 0.0)
    ah_ref[...] = ah
    mv_ref[...] = jnp.dot(pm_ref[...], ah, preferred_element_type=F32)


_tc_final = pl.pallas_call(
    _tc_final_body,
    grid=(N_PAD // ATF,),
    in_specs=[
        pl.BlockSpec((ATF * MAX_NB, HW), lambda i: (i, 0)),
        pl.BlockSpec((ATF, FA), lambda i: (i, 0)),
        pl.BlockSpec((ATF, ATF * MAX_NB), lambda i: (0, 0)),
        pl.BlockSpec((ATF // APM, ATF), lambda i: (0, 0)),
        pl.BlockSpec((FA, H), lambda i: (0, 0)),
        pl.BlockSpec((HW, H), lambda i: (0, 0)),
        pl.BlockSpec((HW, H), lambda i: (0, 0)),
    ],
    out_specs=[
        pl.BlockSpec((ATF, H), lambda i: (i, 0)),
        pl.BlockSpec((ATF // APM, H), lambda i: (i, 0)),
    ],
    out_shape=[
        jax.ShapeDtypeStruct((N_PAD, H), F32),
        jax.ShapeDtypeStruct((N_PAD // APM, H), F32),
    ],
)


def kernel(f_atoms, f_bonds, f_mols, a2b, b2a, b2revb, atoms_per_mol, W_i, W_h, W_o):
    del f_mols, atoms_per_mol
    a2b_flat = jnp.concatenate(
        [a2b, jnp.zeros((N_PAD - N + A_FAST, MAX_NB), jnp.int32)], axis=0
    ).reshape(-1)
    f_atoms_pad = jnp.concatenate(
        [f_atoms, jnp.zeros((N_PAD - N, FA), F32)], axis=0)
    wh_top = W_h[:HW].astype(BF16)
    wh_bot = W_h[HW:].astype(BF16)
    gsum = lax.broadcasted_iota(jnp.int32, (AT, AT * MAX_NB), 1) // MAX_NB
    p_sum = (gsum == lax.broadcasted_iota(jnp.int32, (AT, AT * MAX_NB), 0)
             ).astype(BF16)
    gfin = lax.broadcasted_iota(jnp.int32, (ATF, ATF * MAX_NB), 1) // MAX_NB
    p_fin = (gfin == lax.broadcasted_iota(jnp.int32, (ATF, ATF * MAX_NB), 0)
             ).astype(BF16)
    gmol = lax.broadcasted_iota(jnp.int32, (ATF // APM, ATF), 1) // APM
    p_mol = jnp.where(
        gmol == lax.broadcasted_iota(jnp.int32, (ATF // APM, ATF), 0),
        1.0 / APM, 0.0).astype(F32)

    inpp, msgp = _mm_init(f_bonds, W_i)
    for _ in range(DEPTH - 1):
        nei = _sc_nei(msgp, a2b_flat)
        r = _sc_gather(msgp, b2revb)
        amsgp = _tc_sum(nei, p_sum)
        t = _sc_gather(amsgp, b2a)
        msgp = _mm_iter(t, r, inpp, wh_top, wh_bot)
    nei = _sc_nei(msgp, a2b_flat)
    atom_h, mol_vecs = _tc_final(
        nei, f_atoms_pad, p_fin, p_mol, W_o[:FA], W_o[FA:FA + HW],
        W_o[FA + HW:])
    return (mol_vecs[:N_MOLS], atom_h[:N])
